# baseline (speedup 1.0000x reference)
import functools

import jax
import jax.numpy as jnp
from jax import lax
from jax.experimental import pallas as pl
from jax.experimental.pallas import tpu as pltpu
from jax.experimental.pallas import tpu_sc as plsc

_NC = 2
_NS = 16
_L = 16


def _make_sc_kernel(B, F):
    NW = _NC * _NS
    assert B % (8 * NW) == 0
    bpw = B // NW
    mesh = plsc.VectorSubcoreMesh(core_axis_name="c", subcore_axis_name="s")

    @functools.partial(
        pl.kernel,
        mesh=mesh,
        out_type=jax.ShapeDtypeStruct((B,), jnp.float32),
        compiler_params=pltpu.CompilerParams(
            needs_layout_passes=False, use_tc_tiling_on_sc=False
        ),
        scratch_types=[
            pltpu.VMEM((bpw,), jnp.int32),
            pltpu.VMEM((bpw,), jnp.int32),
            pltpu.VMEM((bpw,), jnp.int32),
            pltpu.VMEM((bpw, F), jnp.float32),
            pltpu.VMEM((bpw, F), jnp.float32),
            pltpu.VMEM((bpw, F), jnp.float32),
            pltpu.VMEM((bpw,), jnp.float32),
            pltpu.SemaphoreType.DMA,
        ],
    )
    def sc_kernel(user_h, pos_h, neg_h, uw_h, iw_h, out_h,
                  uidx, pidx, nidx, ur, pr, nr, ov, sem):
        wid = lax.axis_index("s") * _NC + lax.axis_index("c")
        base = wid * bpw

        pltpu.sync_copy(user_h.at[pl.ds(base, bpw)], uidx)
        pltpu.sync_copy(pos_h.at[pl.ds(base, bpw)], pidx)
        pltpu.sync_copy(neg_h.at[pl.ds(base, bpw)], nidx)

        cu = pltpu.async_copy(uw_h.at[uidx], ur, sem)
        cp = pltpu.async_copy(iw_h.at[pidx], pr, sem)
        cn = pltpu.async_copy(iw_h.at[nidx], nr, sem)
        cu.wait()
        cp.wait()
        cn.wait()

        lanes = lax.iota(jnp.int32, _L)

        def group(g, carry):
            rows = g * _L + lanes
            acc = jnp.zeros((_L,), jnp.float32)
            for f in range(F):
                col = jnp.full((_L,), f, jnp.int32)
                uu = plsc.load_gather(ur, [rows, col])
                pp = plsc.load_gather(pr, [rows, col])
                nn = plsc.load_gather(nr, [rows, col])
                acc = acc + uu * (pp - nn)
            sig = 1.0 / (1.0 + jnp.exp(-acc))
            ov[pl.ds(g * _L, _L)] = 2.0 - sig
            return carry

        lax.fori_loop(0, bpw // _L, group, 0)
        pltpu.sync_copy(ov, out_h.at[pl.ds(base, bpw)])

    return sc_kernel


def kernel(user, posItem, negItem, user_W, item_W):
    B = user.shape[0]
    F = user_W.shape[1]
    out = _make_sc_kernel(B, F)(user, posItem, negItem, user_W, item_W)
    return out.reshape(-1, 1)
